# baseline (speedup 1.0000x reference)
import functools

import jax
import jax.numpy as jnp
from jax import lax
from jax.experimental import pallas as pl
from jax.experimental.pallas import tpu as pltpu
from jax.experimental.pallas import tpu_sc as plsc

NUM_CORES = 2
NUM_SUBCORES = 16
NW = NUM_CORES * NUM_SUBCORES
NBUF = 4


@jax.jit
def _embed(id2, W):
    B, S = id2.shape
    D = W.shape[1]
    seq_per_w = B // NW
    n_outer = seq_per_w // NBUF
    assert seq_per_w % NBUF == 0 and n_outer >= 2
    mesh = plsc.VectorSubcoreMesh(
        core_axis_name="c", subcore_axis_name="s",
        num_cores=NUM_CORES, num_subcores=NUM_SUBCORES)

    @functools.partial(
        pl.kernel,
        mesh=mesh,
        out_type=jax.ShapeDtypeStruct((B, S, D), jnp.float32),
        scratch_types=[
            pltpu.VMEM((seq_per_w, S), jnp.int32),
            pltpu.VMEM((NBUF, S, D), jnp.float32),
        ] + [pltpu.SemaphoreType.DMA] * (2 * NBUF),
        compiler_params=pltpu.CompilerParams(use_tc_tiling_on_sc=False),
    )
    def k(table_hbm, id_hbm, out_hbm, idx_v, bufs, *sems):
        gsem = sems[:NBUF]
        ssem = sems[NBUF:]
        wid = lax.axis_index("s") * NUM_CORES + lax.axis_index("c")
        seq0 = wid * seq_per_w
        pltpu.sync_copy(id_hbm.at[pl.ds(seq0, seq_per_w)], idx_v)

        def gather(j, b):
            pltpu.async_copy(table_hbm.at[idx_v.at[j]], bufs.at[b], gsem[b])

        def store(j, b):
            pltpu.async_copy(bufs.at[b], out_hbm.at[seq0 + j], ssem[b])

        def wait_gather(j, b):
            pltpu.make_async_copy(
                table_hbm.at[idx_v.at[j]], bufs.at[b], gsem[b]).wait()

        def wait_store(j, b):
            pltpu.make_async_copy(
                bufs.at[b], out_hbm.at[seq0 + j], ssem[b]).wait()

        for b in range(NBUF):
            gather(b, b)

        def body(g, carry):
            for b in range(NBUF):
                j = g * NBUF + b
                wait_gather(j, b)
                store(j, b)
                wait_store(j, b)
                gather(j + NBUF, b)
            return carry

        lax.fori_loop(0, n_outer - 1, body, 0)

        for b in range(NBUF):
            j = (n_outer - 1) * NBUF + b
            wait_gather(j, b)
            store(j, b)
        for b in range(NBUF):
            j = (n_outer - 1) * NBUF + b
            wait_store(j, b)

    return k(W, id2)


def kernel(id, W):
    return _embed(id.astype(jnp.int32), W)
